# Initial kernel scaffold; baseline (speedup 1.0000x reference)
import jax
import jax.numpy as jnp
from jax.experimental import pallas as pl


def kernel(x, edge_index, W0, W1):
    raise NotImplementedError("write your pallas kernel here")


# speedup vs baseline: 28.6771x; 28.6771x over previous
import functools

import jax
import jax.numpy as jnp
from jax import lax
from jax.experimental import pallas as pl
from jax.experimental.pallas import tpu as pltpu
from jax.experimental.pallas import tpu_sc as plsc

N = 10000
E = 320000
D = 128
H = 16

NW = 32
CH = 80
NCH = E // (NW * CH)
ROWS_W = N // 16
DEG_W = 632
N_DEG = 16 * DEG_W

_mesh = plsc.VectorSubcoreMesh(core_axis_name="c", subcore_axis_name="s")
_sc_params = pltpu.CompilerParams(use_tc_tiling_on_sc=False)


@functools.partial(
    pl.kernel,
    mesh=_mesh,
    out_type=jax.ShapeDtypeStruct((NW, 1, DEG_W), jnp.float32),
    compiler_params=_sc_params,
    scratch_types=[
        pltpu.VMEM((NCH, CH), jnp.int32),
        pltpu.VMEM((CH,), jnp.float32),
        pltpu.VMEM((DEG_W,), jnp.float32),
        pltpu.VMEM_SHARED((N_DEG,), jnp.float32),
    ],
)
def _sc_deg(dst_hbm, ones_hbm, z_hbm, out_hbm, didx, onesv, zv, acc):
    c = lax.axis_index("c")
    s = lax.axis_index("s")
    w = c * 16 + s
    pltpu.sync_copy(z_hbm, zv)
    pltpu.sync_copy(zv, acc.at[pl.ds(s * DEG_W, DEG_W)])
    pltpu.sync_copy(ones_hbm, onesv)
    pltpu.sync_copy(dst_hbm.at[w], didx)
    plsc.subcore_barrier()

    def body(j, carry):
        pltpu.sync_copy(onesv, acc.at[didx.at[j]], add=True)
        return carry

    lax.fori_loop(0, NCH, body, 0)
    plsc.subcore_barrier()
    pltpu.sync_copy(acc.at[pl.ds(s * DEG_W, DEG_W)], zv)
    pltpu.sync_copy(zv, out_hbm.at[w, 0])


@functools.partial(
    pl.kernel,
    mesh=_mesh,
    out_type=jax.ShapeDtypeStruct((NW, ROWS_W, H), jnp.float32),
    compiler_params=_sc_params,
    scratch_types=[
        pltpu.VMEM((NCH, CH), jnp.int32),
        pltpu.VMEM((NCH, CH), jnp.int32),
        pltpu.VMEM((CH, H), jnp.float32),
        pltpu.VMEM((ROWS_W, H), jnp.float32),
        pltpu.VMEM_SHARED((N, H), jnp.float32),
        pltpu.SemaphoreType.DMA,
    ],
)
def _sc_msg(g_hbm, src_hbm, dst_hbm, z_hbm, out_hbm, sidx, didx, rows, zv, acc, sem):
    c = lax.axis_index("c")
    s = lax.axis_index("s")
    w = c * 16 + s
    pltpu.sync_copy(z_hbm, zv)
    pltpu.sync_copy(zv, acc.at[pl.ds(s * ROWS_W, ROWS_W)])
    pltpu.sync_copy(src_hbm.at[w], sidx)
    pltpu.sync_copy(dst_hbm.at[w], didx)
    plsc.subcore_barrier()

    def body(j, carry):
        pltpu.async_copy(g_hbm.at[sidx.at[j]], rows, sem).wait()
        pltpu.sync_copy(rows, acc.at[didx.at[j]], add=True)
        return carry

    lax.fori_loop(0, NCH, body, 0)
    plsc.subcore_barrier()
    pltpu.sync_copy(acc.at[pl.ds(s * ROWS_W, ROWS_W)], zv)
    pltpu.sync_copy(zv, out_hbm.at[w])


_BLK = 1000
_GRID = N // _BLK


def _tc_a_body(x_ref, w0_ref, dp_ref, g_ref, dinv_ref):
    deg = dp_ref[0] + dp_ref[1] + 1.0
    dinv = lax.rsqrt(deg)
    h = jnp.dot(x_ref[...], w0_ref[...], preferred_element_type=jnp.float32)
    g_ref[...] = h * dinv
    dinv_ref[...] = dinv


def _tc_a(x, W0, degparts):
    dp = degparts.reshape(2, N, 1)
    return pl.pallas_call(
        _tc_a_body,
        grid=(_GRID,),
        in_specs=[
            pl.BlockSpec((_BLK, D), lambda i: (i, 0)),
            pl.BlockSpec((D, H), lambda i: (0, 0)),
            pl.BlockSpec((2, _BLK, 1), lambda i: (0, i, 0)),
        ],
        out_specs=[
            pl.BlockSpec((_BLK, H), lambda i: (i, 0)),
            pl.BlockSpec((_BLK, 1), lambda i: (i, 0)),
        ],
        out_shape=[
            jax.ShapeDtypeStruct((N, H), jnp.float32),
            jax.ShapeDtypeStruct((N, 1), jnp.float32),
        ],
    )(x, W0, dp)


def _tc_b_body(p_ref, g_ref, dinv_ref, w1_ref, g1_ref):
    dinv = dinv_ref[...]
    o1 = dinv * (p_ref[0] + p_ref[1] + g_ref[...])
    a = jnp.maximum(o1, 0.0)
    h1 = jnp.dot(a, w1_ref[...], preferred_element_type=jnp.float32)
    g1_ref[...] = h1 * dinv


def _tc_b(parts, g, dinv, W1):
    return pl.pallas_call(
        _tc_b_body,
        grid=(_GRID,),
        in_specs=[
            pl.BlockSpec((2, _BLK, H), lambda i: (0, i, 0)),
            pl.BlockSpec((_BLK, H), lambda i: (i, 0)),
            pl.BlockSpec((_BLK, 1), lambda i: (i, 0)),
            pl.BlockSpec((H, H), lambda i: (0, 0)),
        ],
        out_specs=pl.BlockSpec((_BLK, H), lambda i: (i, 0)),
        out_shape=jax.ShapeDtypeStruct((N, H), jnp.float32),
    )(parts, g, dinv, W1)


def _tc_c_body(p_ref, g1_ref, dinv_ref, out_ref):
    o2 = dinv_ref[...] * (p_ref[0] + p_ref[1] + g1_ref[...])
    out_ref[...] = jnp.exp(o2) + 1.0


def _tc_c(parts, g1, dinv):
    return pl.pallas_call(
        _tc_c_body,
        grid=(_GRID,),
        in_specs=[
            pl.BlockSpec((2, _BLK, H), lambda i: (0, i, 0)),
            pl.BlockSpec((_BLK, H), lambda i: (i, 0)),
            pl.BlockSpec((_BLK, 1), lambda i: (i, 0)),
        ],
        out_specs=pl.BlockSpec((_BLK, H), lambda i: (i, 0)),
        out_shape=jax.ShapeDtypeStruct((N, H), jnp.float32),
    )(parts, g1, dinv)


def kernel(x, edge_index, W0, W1):
    src3 = edge_index[0].reshape(NW, NCH, CH)
    dst3 = edge_index[1].reshape(NW, NCH, CH)
    zrows = jnp.zeros((ROWS_W, H), jnp.float32)
    zvec = jnp.zeros((DEG_W,), jnp.float32)
    ones = jnp.ones((CH,), jnp.float32)

    degparts = _sc_deg(dst3, ones, zvec).reshape(2, N_DEG)[:, :N]
    g, dinv = _tc_a(x, W0, degparts)
    parts1 = _sc_msg(g, src3, dst3, zrows).reshape(2, N, H)
    g1 = _tc_b(parts1, g, dinv, W1)
    parts2 = _sc_msg(g1, src3, dst3, zrows).reshape(2, N, H)
    return _tc_c(parts2, g1, dinv)


# speedup vs baseline: 56.4553x; 1.9687x over previous
import functools

import jax
import jax.numpy as jnp
from jax import lax
from jax.experimental import pallas as pl
from jax.experimental.pallas import tpu as pltpu
from jax.experimental.pallas import tpu_sc as plsc

N = 10000
E = 320000
D = 128
H = 16

NW = 32
CH = 80
NCH = E // (NW * CH)
CHU = 2000
NCHU = E // (NW * CHU)
ROWS_W = N // 16
DEG_W = 632
N_DEG = 16 * DEG_W

_mesh = plsc.VectorSubcoreMesh(core_axis_name="c", subcore_axis_name="s")
_sc_params = pltpu.CompilerParams(use_tc_tiling_on_sc=False)


@functools.partial(
    pl.kernel,
    mesh=_mesh,
    out_type=jax.ShapeDtypeStruct((NW, 1, DEG_W), jnp.float32),
    compiler_params=_sc_params,
    scratch_types=[
        pltpu.VMEM((NCH, CH), jnp.int32),
        pltpu.VMEM((CH,), jnp.float32),
        pltpu.VMEM((DEG_W,), jnp.float32),
        pltpu.VMEM_SHARED((N_DEG,), jnp.float32),
    ],
)
def _sc_deg(dst_hbm, ones_hbm, z_hbm, out_hbm, didx, onesv, zv, acc):
    c = lax.axis_index("c")
    s = lax.axis_index("s")
    w = c * 16 + s
    pltpu.sync_copy(z_hbm, zv)
    pltpu.sync_copy(zv, acc.at[pl.ds(s * DEG_W, DEG_W)])
    pltpu.sync_copy(ones_hbm, onesv)
    pltpu.sync_copy(dst_hbm.at[w], didx)
    plsc.subcore_barrier()

    def body(j, carry):
        pltpu.sync_copy(onesv, acc.at[didx.at[j]], add=True)
        return carry

    lax.fori_loop(0, NCH, body, 0)
    plsc.subcore_barrier()
    pltpu.sync_copy(acc.at[pl.ds(s * DEG_W, DEG_W)], zv)
    pltpu.sync_copy(zv, out_hbm.at[w, 0])


@functools.partial(
    pl.kernel,
    mesh=_mesh,
    out_type=jax.ShapeDtypeStruct((NW, ROWS_W, H), jnp.float32),
    compiler_params=_sc_params,
    scratch_types=[
        pltpu.VMEM((NCHU * CHU,), jnp.int32),
        pltpu.VMEM((NCHU * CHU,), jnp.int32),
        pltpu.VMEM((CHU, H), jnp.float32),
        pltpu.VMEM((CHU, H), jnp.float32),
        pltpu.VMEM((ROWS_W, H), jnp.float32),
        pltpu.VMEM_SHARED((N, H), jnp.float32),
        pltpu.SemaphoreType.DMA,
        pltpu.SemaphoreType.DMA,
    ],
)
def _sc_msg(g_hbm, src_hbm, dst_hbm, z_hbm, out_hbm, sidx, didx, rows0, rows1,
            zv, acc, sem0, sem1):
    c = lax.axis_index("c")
    s = lax.axis_index("s")
    w = c * 16 + s
    pltpu.sync_copy(z_hbm, zv)
    pltpu.sync_copy(zv, acc.at[pl.ds(s * ROWS_W, ROWS_W)])
    pltpu.sync_copy(src_hbm.at[w], sidx)
    pltpu.sync_copy(dst_hbm.at[w], didx)
    plsc.subcore_barrier()

    def gather(k, rows, sem):
        return pltpu.async_copy(g_hbm.at[sidx.at[pl.ds(k * CHU, CHU)]],
                                rows, sem)

    def scatter(k, rows):
        pltpu.sync_copy(rows, acc.at[didx.at[pl.ds(k * CHU, CHU)]], add=True)

    gather(0, rows0, sem0).wait()

    def body(p, carry):
        h1 = gather(2 * p + 1, rows1, sem1)
        scatter(2 * p, rows0)
        h1.wait()
        h0 = gather(2 * p + 2, rows0, sem0)
        scatter(2 * p + 1, rows1)
        h0.wait()
        return carry

    lax.fori_loop(0, (NCHU - 1) // 2, body, 0)
    scatter(NCHU - 1, rows0)
    plsc.subcore_barrier()
    pltpu.sync_copy(acc.at[pl.ds(s * ROWS_W, ROWS_W)], zv)
    pltpu.sync_copy(zv, out_hbm.at[w])


_BLK = 1000
_GRID = N // _BLK


def _tc_a_body(x_ref, w0_ref, dp_ref, g_ref, dinv_ref):
    deg = dp_ref[0] + dp_ref[1] + 1.0
    dinv = lax.rsqrt(deg)
    h = jnp.dot(x_ref[...], w0_ref[...], preferred_element_type=jnp.float32)
    g_ref[...] = h * dinv
    dinv_ref[...] = dinv


def _tc_a(x, W0, degparts):
    dp = degparts.reshape(2, N, 1)
    return pl.pallas_call(
        _tc_a_body,
        grid=(_GRID,),
        in_specs=[
            pl.BlockSpec((_BLK, D), lambda i: (i, 0)),
            pl.BlockSpec((D, H), lambda i: (0, 0)),
            pl.BlockSpec((2, _BLK, 1), lambda i: (0, i, 0)),
        ],
        out_specs=[
            pl.BlockSpec((_BLK, H), lambda i: (i, 0)),
            pl.BlockSpec((_BLK, 1), lambda i: (i, 0)),
        ],
        out_shape=[
            jax.ShapeDtypeStruct((N, H), jnp.float32),
            jax.ShapeDtypeStruct((N, 1), jnp.float32),
        ],
    )(x, W0, dp)


def _tc_b_body(p_ref, g_ref, dinv_ref, w1_ref, g1_ref):
    dinv = dinv_ref[...]
    o1 = dinv * (p_ref[0] + p_ref[1] + g_ref[...])
    a = jnp.maximum(o1, 0.0)
    h1 = jnp.dot(a, w1_ref[...], preferred_element_type=jnp.float32)
    g1_ref[...] = h1 * dinv


def _tc_b(parts, g, dinv, W1):
    return pl.pallas_call(
        _tc_b_body,
        grid=(_GRID,),
        in_specs=[
            pl.BlockSpec((2, _BLK, H), lambda i: (0, i, 0)),
            pl.BlockSpec((_BLK, H), lambda i: (i, 0)),
            pl.BlockSpec((_BLK, 1), lambda i: (i, 0)),
            pl.BlockSpec((H, H), lambda i: (0, 0)),
        ],
        out_specs=pl.BlockSpec((_BLK, H), lambda i: (i, 0)),
        out_shape=jax.ShapeDtypeStruct((N, H), jnp.float32),
    )(parts, g, dinv, W1)


def _tc_c_body(p_ref, g1_ref, dinv_ref, out_ref):
    o2 = dinv_ref[...] * (p_ref[0] + p_ref[1] + g1_ref[...])
    out_ref[...] = jnp.exp(o2) + 1.0


def _tc_c(parts, g1, dinv):
    return pl.pallas_call(
        _tc_c_body,
        grid=(_GRID,),
        in_specs=[
            pl.BlockSpec((2, _BLK, H), lambda i: (0, i, 0)),
            pl.BlockSpec((_BLK, H), lambda i: (i, 0)),
            pl.BlockSpec((_BLK, 1), lambda i: (i, 0)),
        ],
        out_specs=pl.BlockSpec((_BLK, H), lambda i: (i, 0)),
        out_shape=jax.ShapeDtypeStruct((N, H), jnp.float32),
    )(parts, g1, dinv)


def kernel(x, edge_index, W0, W1):
    src4 = edge_index[0].reshape(NW, NCHU * CHU)
    dst3 = edge_index[1].reshape(NW, NCH, CH)
    dst4 = edge_index[1].reshape(NW, NCHU * CHU)
    zrows = jnp.zeros((ROWS_W, H), jnp.float32)
    zvec = jnp.zeros((DEG_W,), jnp.float32)
    ones = jnp.ones((CH,), jnp.float32)

    degparts = _sc_deg(dst3, ones, zvec).reshape(2, N_DEG)[:, :N]
    g, dinv = _tc_a(x, W0, degparts)
    parts1 = _sc_msg(g, src4, dst4, zrows).reshape(2, N, H)
    g1 = _tc_b(parts1, g, dinv, W1)
    parts2 = _sc_msg(g1, src4, dst4, zrows).reshape(2, N, H)
    return _tc_c(parts2, g1, dinv)


# speedup vs baseline: 60.5568x; 1.0727x over previous
import functools

import jax
import jax.numpy as jnp
from jax import lax
from jax.experimental import pallas as pl
from jax.experimental.pallas import tpu as pltpu
from jax.experimental.pallas import tpu_sc as plsc

N = 10000
E = 320000
D = 128
H = 16

NW = 32
CH = 80
NCH = E // (NW * CH)
CHU = 2000
NCHU = E // (NW * CHU)
ROWS_W = N // 16
DEG_W = 632
N_DEG = 16 * DEG_W

_mesh = plsc.VectorSubcoreMesh(core_axis_name="c", subcore_axis_name="s")
_sc_params = pltpu.CompilerParams(use_tc_tiling_on_sc=False)


@functools.partial(
    pl.kernel,
    mesh=_mesh,
    out_type=jax.ShapeDtypeStruct((NW, 1, DEG_W), jnp.float32),
    compiler_params=_sc_params,
    scratch_types=[
        pltpu.VMEM((E // NW,), jnp.int32),
        pltpu.VMEM((E // NW,), jnp.float32),
        pltpu.VMEM((DEG_W,), jnp.float32),
        pltpu.VMEM_SHARED((N_DEG,), jnp.float32),
    ],
)
def _sc_deg(dst_hbm, ones_hbm, z_hbm, out_hbm, didx, onesv, zv, acc):
    c = lax.axis_index("c")
    s = lax.axis_index("s")
    w = c * 16 + s
    pltpu.sync_copy(z_hbm, zv)
    pltpu.sync_copy(zv, acc.at[pl.ds(s * DEG_W, DEG_W)])
    pltpu.sync_copy(ones_hbm, onesv)
    pltpu.sync_copy(dst_hbm.at[w], didx)
    plsc.subcore_barrier()
    pltpu.sync_copy(onesv, acc.at[didx], add=True)
    plsc.subcore_barrier()
    pltpu.sync_copy(acc.at[pl.ds(s * DEG_W, DEG_W)], zv)
    pltpu.sync_copy(zv, out_hbm.at[w, 0])


@functools.partial(
    pl.kernel,
    mesh=_mesh,
    out_type=jax.ShapeDtypeStruct((NW, ROWS_W, H), jnp.float32),
    compiler_params=_sc_params,
    scratch_types=[
        pltpu.VMEM((NCHU * CHU,), jnp.int32),
        pltpu.VMEM((NCHU * CHU,), jnp.int32),
        pltpu.VMEM((CHU, H), jnp.float32),
        pltpu.VMEM((CHU, H), jnp.float32),
        pltpu.VMEM((ROWS_W, H), jnp.float32),
        pltpu.VMEM_SHARED((N, H), jnp.float32),
        pltpu.SemaphoreType.DMA,
        pltpu.SemaphoreType.DMA,
    ],
)
def _sc_msg(g_hbm, src_hbm, dst_hbm, z_hbm, out_hbm, sidx, didx, rows0, rows1,
            zv, acc, sem0, sem1):
    c = lax.axis_index("c")
    s = lax.axis_index("s")
    w = c * 16 + s
    pltpu.sync_copy(z_hbm, zv)
    pltpu.sync_copy(zv, acc.at[pl.ds(s * ROWS_W, ROWS_W)])
    pltpu.sync_copy(src_hbm.at[w], sidx)
    pltpu.sync_copy(dst_hbm.at[w], didx)
    plsc.subcore_barrier()

    def gather(k, rows, sem):
        return pltpu.async_copy(g_hbm.at[sidx.at[pl.ds(k * CHU, CHU)]],
                                rows, sem)

    def scatter(k, rows):
        pltpu.sync_copy(rows, acc.at[didx.at[pl.ds(k * CHU, CHU)]], add=True)

    gather(0, rows0, sem0).wait()

    def body(p, carry):
        h1 = gather(2 * p + 1, rows1, sem1)
        scatter(2 * p, rows0)
        h1.wait()
        h0 = gather(2 * p + 2, rows0, sem0)
        scatter(2 * p + 1, rows1)
        h0.wait()
        return carry

    lax.fori_loop(0, (NCHU - 1) // 2, body, 0)
    scatter(NCHU - 1, rows0)
    plsc.subcore_barrier()
    pltpu.sync_copy(acc.at[pl.ds(s * ROWS_W, ROWS_W)], zv)
    pltpu.sync_copy(zv, out_hbm.at[w])


def _tc_a_body(x_ref, w0_ref, dp_ref, g_ref, dinv_ref):
    deg = dp_ref[0] + dp_ref[1] + 1.0
    dinv = lax.rsqrt(deg)
    h = jnp.dot(x_ref[...], w0_ref[...], preferred_element_type=jnp.float32)
    g_ref[...] = h * dinv
    dinv_ref[...] = dinv


def _tc_a(x, W0, degparts):
    dp = degparts.reshape(2, N, 1)
    return pl.pallas_call(
        _tc_a_body,
        out_shape=[
            jax.ShapeDtypeStruct((N, H), jnp.float32),
            jax.ShapeDtypeStruct((N, 1), jnp.float32),
        ],
    )(x, W0, dp)


def _tc_b_body(p_ref, g_ref, dinv_ref, w1_ref, g1_ref):
    dinv = dinv_ref[...]
    o1 = dinv * (p_ref[0] + p_ref[1] + g_ref[...])
    a = jnp.maximum(o1, 0.0)
    h1 = jnp.dot(a, w1_ref[...], preferred_element_type=jnp.float32)
    g1_ref[...] = h1 * dinv


def _tc_b(parts, g, dinv, W1):
    return pl.pallas_call(
        _tc_b_body,
        out_shape=jax.ShapeDtypeStruct((N, H), jnp.float32),
    )(parts, g, dinv, W1)


def _tc_c_body(p_ref, g1_ref, dinv_ref, out_ref):
    o2 = dinv_ref[...] * (p_ref[0] + p_ref[1] + g1_ref[...])
    out_ref[...] = jnp.exp(o2) + 1.0


def _tc_c(parts, g1, dinv):
    return pl.pallas_call(
        _tc_c_body,
        out_shape=jax.ShapeDtypeStruct((N, H), jnp.float32),
    )(parts, g1, dinv)


def kernel(x, edge_index, W0, W1):
    src4 = edge_index[0].reshape(NW, NCHU * CHU)
    dst4 = edge_index[1].reshape(NW, NCHU * CHU)
    zrows = jnp.zeros((ROWS_W, H), jnp.float32)
    zvec = jnp.zeros((DEG_W,), jnp.float32)
    ones = jnp.ones((E // NW,), jnp.float32)

    degparts = _sc_deg(dst4, ones, zvec).reshape(2, N_DEG)[:, :N]
    g, dinv = _tc_a(x, W0, degparts)
    parts1 = _sc_msg(g, src4, dst4, zrows).reshape(2, N, H)
    g1 = _tc_b(parts1, g, dinv, W1)
    parts2 = _sc_msg(g1, src4, dst4, zrows).reshape(2, N, H)
    return _tc_c(parts2, g1, dinv)
